# baseline (device time: 151539 ns/iter reference)
import jax
import jax.numpy as jnp
from jax import lax
from jax.experimental import pallas as pl
from jax.experimental.pallas import tpu as pltpu

N_DEV = 32
SQ = 256
D = 1024
H = 8
DH = 128
C = SQ // N_DEV
SCALE = 0.08838834764831843


def _allreduce(partial):
    m, n = partial.shape

    def body(in_ref, out_ref, comm_ref, rs_send, rs_recv, ag_send, ag_recv):
        me = lax.axis_index("i")
        left = lax.rem(me + N_DEV - 1, N_DEV)
        right = lax.rem(me + 1, N_DEV)

        barrier_sem = pltpu.get_barrier_semaphore()
        for nbr in (left, right):
            pl.semaphore_signal(
                barrier_sem, inc=1,
                device_id=(nbr,), device_id_type=pl.DeviceIdType.MESH,
            )
        pl.semaphore_wait(barrier_sem, 2)

        out_ref[...] = in_ref[...]

        for s in range(N_DEV - 1):
            send_chunk = lax.bitwise_and(me - s, N_DEV - 1)
            rdma = pltpu.make_async_remote_copy(
                src_ref=out_ref.at[pl.ds(send_chunk * C, C), :],
                dst_ref=comm_ref.at[s],
                send_sem=rs_send.at[s],
                recv_sem=rs_recv.at[s],
                device_id=(right,),
                device_id_type=pl.DeviceIdType.MESH,
            )
            rdma.start()
            rdma.wait()
            recv_chunk = lax.bitwise_and(me - s - 1, N_DEV - 1)
            r0 = recv_chunk * C
            out_ref[pl.ds(r0, C), :] = out_ref[pl.ds(r0, C), :] + comm_ref[s]

        for s in range(N_DEV - 1):
            chunk = lax.bitwise_and(me + 1 - s, N_DEV - 1)
            r0 = chunk * C
            rdma = pltpu.make_async_remote_copy(
                src_ref=out_ref.at[pl.ds(r0, C), :],
                dst_ref=out_ref.at[pl.ds(r0, C), :],
                send_sem=ag_send.at[s],
                recv_sem=ag_recv.at[s],
                device_id=(right,),
                device_id_type=pl.DeviceIdType.MESH,
            )
            rdma.start()
            rdma.wait()

    return pl.pallas_call(
        body,
        out_shape=jax.ShapeDtypeStruct((m, n), jnp.float32),
        in_specs=[pl.BlockSpec(memory_space=pltpu.VMEM)],
        out_specs=pl.BlockSpec(memory_space=pltpu.VMEM),
        scratch_shapes=[
            pltpu.VMEM((N_DEV - 1, C, n), jnp.float32),
            pltpu.SemaphoreType.DMA((N_DEV - 1,)),
            pltpu.SemaphoreType.DMA((N_DEV - 1,)),
            pltpu.SemaphoreType.DMA((N_DEV - 1,)),
            pltpu.SemaphoreType.DMA((N_DEV - 1,)),
        ],
        compiler_params=pltpu.CompilerParams(collective_id=0),
    )(partial)


def kernel(x, Wq, Wo, Wk, Wv):
    bf = jnp.bfloat16
    xb = x.reshape(SQ, D).astype(bf)
    q = jnp.dot(xb, Wq.astype(bf), preferred_element_type=jnp.float32)
    k = jnp.dot(xb, Wk.astype(bf), preferred_element_type=jnp.float32)
    v = jnp.dot(xb, Wv.astype(bf), preferred_element_type=jnp.float32)
    q = q.reshape(SQ, H, DH)
    k = k.reshape(SQ, H, DH)
    v = v.reshape(SQ, H, DH)
    s = jnp.einsum(
        "ihd,jhd->hij", q.astype(bf), k.astype(bf),
        preferred_element_type=jnp.float32,
    ) * SCALE
    p = jax.nn.softmax(s, axis=-1)
    o = jnp.einsum(
        "hij,jhd->ihd", p.astype(bf), v.astype(bf),
        preferred_element_type=jnp.float32,
    )
    partial = jnp.dot(
        o.reshape(SQ, H * DH).astype(bf), Wo.astype(bf),
        preferred_element_type=jnp.float32,
    )
    out = _allreduce(partial)
    return out.reshape(1, SQ, D)


# device time: 57058 ns/iter; 2.6559x vs baseline; 2.6559x over previous
import jax
import jax.numpy as jnp
from jax import lax
from jax.experimental import pallas as pl
from jax.experimental.pallas import tpu as pltpu

N_DEV = 32
SQ = 256
D = 1024
H = 8
DH = 128
C = SQ // N_DEV
SCALE = 0.08838834764831843


LOG_N = 5
_RS_REGION = [0, 128, 192, 224, 240]


def _allreduce(partial):
    m, n = partial.shape

    def body(in_ref, out_ref, comm_ref, rs_send, rs_recv, ag_send, ag_recv):
        me = lax.axis_index("i")

        barrier_sem = pltpu.get_barrier_semaphore()
        for k in range(LOG_N):
            pl.semaphore_signal(
                barrier_sem, inc=1,
                device_id=(me ^ (1 << k),),
                device_id_type=pl.DeviceIdType.MESH,
            )
        pl.semaphore_wait(barrier_sem, LOG_N)

        out_ref[...] = in_ref[...]

        o = me * 0
        for k in range(LOG_N):
            sz = m >> (k + 1)
            bit = (me >> k) & 1
            send_off = o + (1 - bit) * sz
            keep_off = o + bit * sz
            roff = _RS_REGION[k]
            rdma = pltpu.make_async_remote_copy(
                src_ref=out_ref.at[pl.ds(send_off, sz), :],
                dst_ref=comm_ref.at[pl.ds(roff, sz), :],
                send_sem=rs_send.at[k],
                recv_sem=rs_recv.at[k],
                device_id=(me ^ (1 << k),),
                device_id_type=pl.DeviceIdType.MESH,
            )
            rdma.start()
            rdma.wait()
            out_ref[pl.ds(keep_off, sz), :] = (
                out_ref[pl.ds(keep_off, sz), :] + comm_ref[pl.ds(roff, sz), :]
            )
            o = keep_off

        for k in range(LOG_N - 1, -1, -1):
            sz = m >> (k + 1)
            bit = (me >> k) & 1
            rdma = pltpu.make_async_remote_copy(
                src_ref=out_ref.at[pl.ds(o, sz), :],
                dst_ref=out_ref.at[pl.ds(o, sz), :],
                send_sem=ag_send.at[k],
                recv_sem=ag_recv.at[k],
                device_id=(me ^ (1 << k),),
                device_id_type=pl.DeviceIdType.MESH,
            )
            rdma.start()
            rdma.wait()
            o = o - bit * sz

    return pl.pallas_call(
        body,
        out_shape=jax.ShapeDtypeStruct((m, n), jnp.float32),
        in_specs=[pl.BlockSpec(memory_space=pltpu.VMEM)],
        out_specs=pl.BlockSpec(memory_space=pltpu.VMEM),
        scratch_shapes=[
            pltpu.VMEM((248, n), jnp.float32),
            pltpu.SemaphoreType.DMA((LOG_N,)),
            pltpu.SemaphoreType.DMA((LOG_N,)),
            pltpu.SemaphoreType.DMA((LOG_N,)),
            pltpu.SemaphoreType.DMA((LOG_N,)),
        ],
        compiler_params=pltpu.CompilerParams(collective_id=0),
    )(partial)


def kernel(x, Wq, Wo, Wk, Wv):
    bf = jnp.bfloat16
    xb = x.reshape(SQ, D).astype(bf)
    q = jnp.dot(xb, Wq.astype(bf), preferred_element_type=jnp.float32)
    k = jnp.dot(xb, Wk.astype(bf), preferred_element_type=jnp.float32)
    v = jnp.dot(xb, Wv.astype(bf), preferred_element_type=jnp.float32)
    q = q.reshape(SQ, H, DH)
    k = k.reshape(SQ, H, DH)
    v = v.reshape(SQ, H, DH)
    s = jnp.einsum(
        "ihd,jhd->hij", q.astype(bf), k.astype(bf),
        preferred_element_type=jnp.float32,
    ) * SCALE
    p = jax.nn.softmax(s, axis=-1)
    o = jnp.einsum(
        "hij,jhd->ihd", p.astype(bf), v.astype(bf),
        preferred_element_type=jnp.float32,
    )
    partial = jnp.dot(
        o.reshape(SQ, H * DH).astype(bf), Wo.astype(bf),
        preferred_element_type=jnp.float32,
    )
    out = _allreduce(partial)
    return out.reshape(1, SQ, D)


# device time: 44922 ns/iter; 3.3734x vs baseline; 1.2702x over previous
import jax
import jax.numpy as jnp
from jax import lax
from jax.experimental import pallas as pl
from jax.experimental.pallas import tpu as pltpu

N_DEV = 32
SQ = 256
D = 1024
H = 8
DH = 128
C = SQ // N_DEV
SCALE = 0.08838834764831843


LOG_N = 5
_RS_REGION = [0, 128, 192, 224, 240]


def _allreduce(partial):
    m, n = partial.shape
    bf = jnp.bfloat16

    def body(in_ref, out_ref, sbuf, rs_rbuf, ag_rbuf,
             rs_send, rs_recv, ag_send, ag_recv):
        me = lax.axis_index("i")

        barrier_sem = pltpu.get_barrier_semaphore()
        for k in range(LOG_N):
            pl.semaphore_signal(
                barrier_sem, inc=1,
                device_id=(me ^ (1 << k),),
                device_id_type=pl.DeviceIdType.MESH,
            )
        pl.semaphore_wait(barrier_sem, LOG_N)

        out_ref[...] = in_ref[...]

        o = me * 0
        for k in range(LOG_N):
            sz = m >> (k + 1)
            bit = (me >> k) & 1
            send_off = o + (1 - bit) * sz
            keep_off = o + bit * sz
            roff = _RS_REGION[k]
            sbuf[pl.ds(roff, sz), :] = out_ref[pl.ds(send_off, sz), :].astype(bf)
            rdma = pltpu.make_async_remote_copy(
                src_ref=sbuf.at[pl.ds(roff, sz), :],
                dst_ref=rs_rbuf.at[pl.ds(roff, sz), :],
                send_sem=rs_send.at[k],
                recv_sem=rs_recv.at[k],
                device_id=(me ^ (1 << k),),
                device_id_type=pl.DeviceIdType.MESH,
            )
            rdma.start()
            rdma.wait()
            out_ref[pl.ds(keep_off, sz), :] = (
                out_ref[pl.ds(keep_off, sz), :]
                + rs_rbuf[pl.ds(roff, sz), :].astype(jnp.float32)
            )
            o = keep_off

        for k in range(LOG_N - 1, -1, -1):
            sz = m >> (k + 1)
            bit = (me >> k) & 1
            roff = _RS_REGION[k]
            sbuf[pl.ds(roff, sz), :] = out_ref[pl.ds(o, sz), :].astype(bf)
            rdma = pltpu.make_async_remote_copy(
                src_ref=sbuf.at[pl.ds(roff, sz), :],
                dst_ref=ag_rbuf.at[pl.ds(roff, sz), :],
                send_sem=ag_send.at[k],
                recv_sem=ag_recv.at[k],
                device_id=(me ^ (1 << k),),
                device_id_type=pl.DeviceIdType.MESH,
            )
            rdma.start()
            rdma.wait()
            p_off = pl.multiple_of(o + sz - 2 * bit * sz, 8)
            out_ref[pl.ds(p_off, sz), :] = (
                ag_rbuf[pl.ds(roff, sz), :].astype(jnp.float32)
            )
            o = o - bit * sz

    return pl.pallas_call(
        body,
        out_shape=jax.ShapeDtypeStruct((m, n), jnp.float32),
        in_specs=[pl.BlockSpec(memory_space=pltpu.VMEM)],
        out_specs=pl.BlockSpec(memory_space=pltpu.VMEM),
        scratch_shapes=[
            pltpu.VMEM((248, n), bf),
            pltpu.VMEM((248, n), bf),
            pltpu.VMEM((248, n), bf),
            pltpu.SemaphoreType.DMA((LOG_N,)),
            pltpu.SemaphoreType.DMA((LOG_N,)),
            pltpu.SemaphoreType.DMA((LOG_N,)),
            pltpu.SemaphoreType.DMA((LOG_N,)),
        ],
        compiler_params=pltpu.CompilerParams(collective_id=0),
    )(partial)


def kernel(x, Wq, Wo, Wk, Wv):
    bf = jnp.bfloat16
    xb = x.reshape(SQ, D).astype(bf)
    q = jnp.dot(xb, Wq.astype(bf), preferred_element_type=jnp.float32)
    k = jnp.dot(xb, Wk.astype(bf), preferred_element_type=jnp.float32)
    v = jnp.dot(xb, Wv.astype(bf), preferred_element_type=jnp.float32)
    q = q.reshape(SQ, H, DH)
    k = k.reshape(SQ, H, DH)
    v = v.reshape(SQ, H, DH)
    s = jnp.einsum(
        "ihd,jhd->hij", q.astype(bf), k.astype(bf),
        preferred_element_type=jnp.float32,
    ) * SCALE
    p = jax.nn.softmax(s, axis=-1)
    o = jnp.einsum(
        "hij,jhd->ihd", p.astype(bf), v.astype(bf),
        preferred_element_type=jnp.float32,
    )
    partial = jnp.dot(
        o.reshape(SQ, H * DH).astype(bf), Wo.astype(bf),
        preferred_element_type=jnp.float32,
    )
    out = _allreduce(partial)
    return out.reshape(1, SQ, D)


# device time: 31752 ns/iter; 4.7726x vs baseline; 1.4148x over previous
import jax
import jax.numpy as jnp
from jax import lax
from jax.experimental import pallas as pl
from jax.experimental.pallas import tpu as pltpu

N_DEV = 32
SQ = 256
D = 1024
H = 8
DH = 128
C = SQ // N_DEV
SCALE = 0.08838834764831843


LOG_N = 5
_RS_REGION = [0, 128, 192, 224, 240]


def _allreduce(partial):
    m, n = partial.shape
    bf = jnp.bfloat16

    def body(in_ref, out_ref, sbuf, rs_rbuf, ag_sbuf, ag_rbuf,
             rs_send, rs_recv, ag_send, ag_recv):
        me = lax.axis_index("i")

        barrier_sem = pltpu.get_barrier_semaphore()
        for j in range(N_DEV - 1):
            pl.semaphore_signal(
                barrier_sem, inc=1,
                device_id=((me + 1 + j) & (N_DEV - 1),),
                device_id_type=pl.DeviceIdType.MESH,
            )
        pl.semaphore_wait(barrier_sem, N_DEV - 1)

        sbuf[...] = in_ref[...].astype(bf)

        rs = []
        for j in range(N_DEV - 1):
            p = (me + 1 + j) & (N_DEV - 1)
            rdma = pltpu.make_async_remote_copy(
                src_ref=sbuf.at[pl.ds(pl.multiple_of(p * C, 8), C), :],
                dst_ref=rs_rbuf.at[j],
                send_sem=rs_send.at[j],
                recv_sem=rs_recv.at[j],
                device_id=(p,),
                device_id_type=pl.DeviceIdType.MESH,
            )
            rdma.start()
            rs.append(rdma)
        for rdma in rs:
            rdma.wait()

        my_row = pl.multiple_of(me * C, 8)
        acc = in_ref[pl.ds(my_row, C), :]
        for j in range(N_DEV - 1):
            acc = acc + rs_rbuf[j].astype(jnp.float32)
        out_ref[pl.ds(my_row, C), :] = acc
        ag_sbuf[...] = acc.astype(bf)

        ag = []
        for j in range(N_DEV - 1):
            p = (me + 1 + j) & (N_DEV - 1)
            rdma = pltpu.make_async_remote_copy(
                src_ref=ag_sbuf,
                dst_ref=ag_rbuf.at[j],
                send_sem=ag_send.at[j],
                recv_sem=ag_recv.at[j],
                device_id=(p,),
                device_id_type=pl.DeviceIdType.MESH,
            )
            rdma.start()
            ag.append(rdma)
        for j in range(N_DEV - 1):
            ag[j].wait()
            q = (me - 1 - j) & (N_DEV - 1)
            out_ref[pl.ds(pl.multiple_of(q * C, 8), C), :] = (
                ag_rbuf[j].astype(jnp.float32)
            )

    return pl.pallas_call(
        body,
        out_shape=jax.ShapeDtypeStruct((m, n), jnp.float32),
        in_specs=[pl.BlockSpec(memory_space=pltpu.VMEM)],
        out_specs=pl.BlockSpec(memory_space=pltpu.VMEM),
        scratch_shapes=[
            pltpu.VMEM((m, n), bf),
            pltpu.VMEM((N_DEV - 1, C, n), bf),
            pltpu.VMEM((C, n), bf),
            pltpu.VMEM((N_DEV - 1, C, n), bf),
            pltpu.SemaphoreType.DMA((N_DEV - 1,)),
            pltpu.SemaphoreType.DMA((N_DEV - 1,)),
            pltpu.SemaphoreType.DMA((N_DEV - 1,)),
            pltpu.SemaphoreType.DMA((N_DEV - 1,)),
        ],
        compiler_params=pltpu.CompilerParams(collective_id=0),
    )(partial)


def _allreduce_butterfly(partial):
    m, n = partial.shape
    bf = jnp.bfloat16

    def body(in_ref, out_ref, sbuf, rs_rbuf, ag_rbuf,
             rs_send, rs_recv, ag_send, ag_recv):
        me = lax.axis_index("i")

        barrier_sem = pltpu.get_barrier_semaphore()
        for k in range(LOG_N):
            pl.semaphore_signal(
                barrier_sem, inc=1,
                device_id=(me ^ (1 << k),),
                device_id_type=pl.DeviceIdType.MESH,
            )
        pl.semaphore_wait(barrier_sem, LOG_N)

        out_ref[...] = in_ref[...]

        o = me * 0
        for k in range(LOG_N):
            sz = m >> (k + 1)
            bit = (me >> k) & 1
            send_off = o + (1 - bit) * sz
            keep_off = o + bit * sz
            roff = _RS_REGION[k]
            sbuf[pl.ds(roff, sz), :] = out_ref[pl.ds(send_off, sz), :].astype(bf)
            rdma = pltpu.make_async_remote_copy(
                src_ref=sbuf.at[pl.ds(roff, sz), :],
                dst_ref=rs_rbuf.at[pl.ds(roff, sz), :],
                send_sem=rs_send.at[k],
                recv_sem=rs_recv.at[k],
                device_id=(me ^ (1 << k),),
                device_id_type=pl.DeviceIdType.MESH,
            )
            rdma.start()
            rdma.wait()
            out_ref[pl.ds(keep_off, sz), :] = (
                out_ref[pl.ds(keep_off, sz), :]
                + rs_rbuf[pl.ds(roff, sz), :].astype(jnp.float32)
            )
            o = keep_off

        for k in range(LOG_N - 1, -1, -1):
            sz = m >> (k + 1)
            bit = (me >> k) & 1
            roff = _RS_REGION[k]
            sbuf[pl.ds(roff, sz), :] = out_ref[pl.ds(o, sz), :].astype(bf)
            rdma = pltpu.make_async_remote_copy(
                src_ref=sbuf.at[pl.ds(roff, sz), :],
                dst_ref=ag_rbuf.at[pl.ds(roff, sz), :],
                send_sem=ag_send.at[k],
                recv_sem=ag_recv.at[k],
                device_id=(me ^ (1 << k),),
                device_id_type=pl.DeviceIdType.MESH,
            )
            rdma.start()
            rdma.wait()
            p_off = pl.multiple_of(o + sz - 2 * bit * sz, 8)
            out_ref[pl.ds(p_off, sz), :] = (
                ag_rbuf[pl.ds(roff, sz), :].astype(jnp.float32)
            )
            o = o - bit * sz

    return pl.pallas_call(
        body,
        out_shape=jax.ShapeDtypeStruct((m, n), jnp.float32),
        in_specs=[pl.BlockSpec(memory_space=pltpu.VMEM)],
        out_specs=pl.BlockSpec(memory_space=pltpu.VMEM),
        scratch_shapes=[
            pltpu.VMEM((248, n), bf),
            pltpu.VMEM((248, n), bf),
            pltpu.VMEM((248, n), bf),
            pltpu.SemaphoreType.DMA((LOG_N,)),
            pltpu.SemaphoreType.DMA((LOG_N,)),
            pltpu.SemaphoreType.DMA((LOG_N,)),
            pltpu.SemaphoreType.DMA((LOG_N,)),
        ],
        compiler_params=pltpu.CompilerParams(collective_id=0),
    )(partial)


def kernel(x, Wq, Wo, Wk, Wv):
    bf = jnp.bfloat16
    xb = x.reshape(SQ, D).astype(bf)
    q = jnp.dot(xb, Wq.astype(bf), preferred_element_type=jnp.float32)
    k = jnp.dot(xb, Wk.astype(bf), preferred_element_type=jnp.float32)
    v = jnp.dot(xb, Wv.astype(bf), preferred_element_type=jnp.float32)
    q = q.reshape(SQ, H, DH)
    k = k.reshape(SQ, H, DH)
    v = v.reshape(SQ, H, DH)
    s = jnp.einsum(
        "ihd,jhd->hij", q.astype(bf), k.astype(bf),
        preferred_element_type=jnp.float32,
    ) * SCALE
    p = jax.nn.softmax(s, axis=-1)
    o = jnp.einsum(
        "hij,jhd->ihd", p.astype(bf), v.astype(bf),
        preferred_element_type=jnp.float32,
    )
    partial = jnp.dot(
        o.reshape(SQ, H * DH).astype(bf), Wo.astype(bf),
        preferred_element_type=jnp.float32,
    )
    out = _allreduce(partial)
    return out.reshape(1, SQ, D)
